# Initial kernel scaffold; baseline (speedup 1.0000x reference)
import jax
import jax.numpy as jnp
from jax.experimental import pallas as pl


def kernel(vertices, features, W_e1, b_e1, W_e2, b_e2, W_o1, b_o1, W_o2, b_o2):
    raise NotImplementedError("write your pallas kernel here")


# speedup vs baseline: 2.8497x; 2.8497x over previous
import functools

import numpy as np
import jax
import jax.numpy as jnp
from jax import lax
from jax.experimental import pallas as pl
from jax.experimental.pallas import tpu as pltpu
from jax.experimental.pallas import tpu_sc as plsc

RES = 16
M = RES * RES * RES
N_PTS = 16384
KNN = 16
CIN = 128
PE = 32
GFD = 3 * PE
HID = 256
COUT = 128
SCALER = 8.0

F32 = jnp.float32
I32 = jnp.int32
BIG = 3.0e38
BIGI = 1 << 30


def _grid_constants():
    axes = [np.linspace(-1.0, 1.0, RES, dtype=np.float32)] * 3
    g = np.stack(np.meshgrid(*axes, indexing="ij"), axis=-1).reshape(-1, 3)
    freqs = (2.0 ** np.arange(PE // 2, dtype=np.float32)) * (2.0 * np.pi / 2.0)
    xf = g[..., None] * freqs
    enc = np.concatenate([np.sin(xf), np.cos(xf)], axis=-1).reshape(M, GFD)
    return g.astype(np.float32), enc.astype(np.float32)


_GRID_V, _GRID_FEAT = _grid_constants()
_GRID_S = SCALER * _GRID_V


def _p_body(f_ref, v_ref, wa_ref, wc_ref, p_ref):
    acc = jnp.dot(f_ref[...], wa_ref[...], preferred_element_type=F32)
    v = v_ref[...]
    wc = wc_ref[...]
    for c in range(3):
        acc += v[:, c:c + 1] * (SCALER * wc[c:c + 1, :])
    p_ref[...] = acc


def _q_body(gf_ref, gs_ref, wb_ref, wc_ref, b_ref, q_ref):
    q = jnp.dot(gf_ref[...], wb_ref[...], preferred_element_type=F32) + b_ref[...]
    gs = gs_ref[...]
    wc = wc_ref[...]
    for c in range(3):
        q -= gs[:, c:c + 1] * wc[c:c + 1, :]
    q_ref[...] = q


MB = 256
CHN = 2048
NCH = N_PTS // CHN


def _knn_body(gs_ref, gsb_ref, inT_ref, inTb_ref, nbr_ref, bd_ref, bi_ref):
    ch = pl.program_id(1)
    x = inT_ref[0:1, :] * SCALER
    y = inT_ref[1:2, :] * SCALER
    z = inT_ref[2:3, :] * SCALER
    ssq_in = (x * x + y * y) + z * z
    g = gs_ref[...]
    ssq_out = ((g[:, 0:1] * g[:, 0:1] + g[:, 1:2] * g[:, 1:2])
               + g[:, 2:3] * g[:, 2:3])
    inb = inTb_ref[...] * jnp.bfloat16(SCALER)
    cross = jnp.dot(gsb_ref[...], inb, preferred_element_type=F32)
    d = (ssq_out - 2.0 * cross) + ssq_in

    lane = lax.broadcasted_iota(I32, (MB, CHN), 1)
    col16 = lax.broadcasted_iota(I32, (MB, KNN), 1)
    offs = ch * CHN
    cd = jnp.full((MB, KNN), BIG, F32)
    ci = jnp.zeros((MB, KNN), I32)
    for t in range(KNN):
        m = jnp.min(d, axis=1, keepdims=True)
        am = jnp.min(jnp.where(d == m, lane, BIGI), axis=1, keepdims=True)
        d = jnp.where(lane == am, BIG, d)
        cd = jnp.where(col16 == t, m, cd)
        ci = jnp.where(col16 == t, am + offs, ci)

    @pl.when(ch == 0)
    def _():
        bd_ref[...] = jnp.full((MB, KNN), BIG, F32)
        bi_ref[...] = jnp.zeros((MB, KNN), I32)

    ad = jnp.concatenate([bd_ref[...], cd], axis=1)
    ai = jnp.concatenate([bi_ref[...], ci], axis=1)
    lane32 = lax.broadcasted_iota(I32, (MB, 2 * KNN), 1)
    nd = jnp.zeros((MB, KNN), F32)
    ni = jnp.zeros((MB, KNN), I32)
    for t in range(KNN):
        m = jnp.min(ad, axis=1, keepdims=True)
        am = jnp.min(jnp.where(ad == m, lane32, BIGI), axis=1, keepdims=True)
        sel = lane32 == am
        iv = jnp.min(jnp.where(sel, ai, BIGI), axis=1, keepdims=True)
        ad = jnp.where(sel, BIG, ad)
        nd = jnp.where(col16 == t, m, nd)
        ni = jnp.where(col16 == t, iv, ni)
    bd_ref[...] = nd
    bi_ref[...] = ni

    @pl.when(ch == NCH - 1)
    def _():
        nbr_ref[...] = ni


CMB = 256


def _combine_body(G_ref, q_ref, we2_ref, be2_ref, wo1_ref, bo1_ref,
                  wo2_ref, bo2_ref, out_ref):
    q = q_ref[...]
    acc = jnp.zeros((CMB, HID), F32)
    for k in range(KNN):
        acc += jnp.maximum(G_ref[k] + q, 0.0)
    hbar = acc * (1.0 / KNN)
    red = jnp.dot(hbar, we2_ref[...], preferred_element_type=F32) + be2_ref[...]
    h2 = jnp.maximum(jnp.dot(red, wo1_ref[...], preferred_element_type=F32)
                     + bo1_ref[...], 0.0)
    out_ref[...] = jnp.dot(h2, wo2_ref[...], preferred_element_type=F32) + bo2_ref[...]


SC_CH = 128


def _sc_gather(P, idx2d, n_edges):
    info = plsc.get_sparse_core_info()
    nw = info.num_cores * info.num_subcores
    b_per_w = n_edges // nw
    nch = b_per_w // SC_CH
    mesh = plsc.VectorSubcoreMesh(core_axis_name="c", subcore_axis_name="s")

    @functools.partial(
        pl.kernel, mesh=mesh,
        out_type=jax.ShapeDtypeStruct((n_edges, HID), F32),
        scratch_types=[
            pltpu.VMEM((nch, SC_CH), I32),
            pltpu.VMEM((SC_CH, HID), F32),
            pltpu.VMEM((SC_CH, HID), F32),
            pltpu.SemaphoreType.DMA,
            pltpu.SemaphoreType.DMA,
        ],
    )
    def gather_k(table_hbm, idx_hbm, out_hbm, idx_v, buf0, buf1, sem0, sem1):
        wid = lax.axis_index("s") * info.num_cores + lax.axis_index("c")
        base = wid * b_per_w
        pltpu.sync_copy(idx_hbm.at[pl.ds(wid * nch, nch)], idx_v)
        bufs = (buf0, buf1)
        sems = (sem0, sem1)
        copies = [None, None]
        copies[0] = pltpu.async_copy(table_hbm.at[idx_v.at[0]], buf0, sem0)
        for ch in range(nch):
            cur = ch % 2
            if ch + 1 < nch:
                copies[(ch + 1) % 2] = pltpu.async_copy(
                    table_hbm.at[idx_v.at[ch + 1]], bufs[(ch + 1) % 2],
                    sems[(ch + 1) % 2])
            copies[cur].wait()
            pltpu.sync_copy(bufs[cur], out_hbm.at[pl.ds(base + ch * SC_CH, SC_CH)])

    return gather_k(P, idx2d)


def kernel(vertices, features, W_e1, b_e1, W_e2, b_e2, W_o1, b_o1, W_o2, b_o2):
    W1a = W_e1[:CIN]
    W1b = W_e1[CIN:CIN + GFD]
    W1c = W_e1[CIN + GFD:]

    PB = 2048
    P = pl.pallas_call(
        _p_body,
        grid=(N_PTS // PB,),
        in_specs=[
            pl.BlockSpec((PB, CIN), lambda i: (i, 0)),
            pl.BlockSpec((PB, 3), lambda i: (i, 0)),
            pl.BlockSpec((CIN, HID), lambda i: (0, 0)),
            pl.BlockSpec((3, HID), lambda i: (0, 0)),
        ],
        out_specs=pl.BlockSpec((PB, HID), lambda i: (i, 0)),
        out_shape=jax.ShapeDtypeStruct((N_PTS, HID), F32),
    )(features, vertices, W1a, W1c)

    Q = pl.pallas_call(
        _q_body,
        grid=(1,),
        in_specs=[
            pl.BlockSpec((M, GFD), lambda i: (0, 0)),
            pl.BlockSpec((M, 3), lambda i: (0, 0)),
            pl.BlockSpec((GFD, HID), lambda i: (0, 0)),
            pl.BlockSpec((3, HID), lambda i: (0, 0)),
            pl.BlockSpec((1, HID), lambda i: (0, 0)),
        ],
        out_specs=pl.BlockSpec((M, HID), lambda i: (0, 0)),
        out_shape=jax.ShapeDtypeStruct((M, HID), F32),
    )(jnp.asarray(_GRID_FEAT), jnp.asarray(_GRID_S), W1b, W1c,
      b_e1.reshape(1, HID))

    in_T = vertices.T
    in_Tb = in_T.astype(jnp.bfloat16)
    nbr = pl.pallas_call(
        _knn_body,
        grid=(M // MB, NCH),
        in_specs=[
            pl.BlockSpec((MB, 3), lambda i, j: (i, 0)),
            pl.BlockSpec((MB, 3), lambda i, j: (i, 0)),
            pl.BlockSpec((3, CHN), lambda i, j: (0, j)),
            pl.BlockSpec((3, CHN), lambda i, j: (0, j)),
        ],
        out_specs=pl.BlockSpec((MB, KNN), lambda i, j: (i, 0)),
        out_shape=jax.ShapeDtypeStruct((M, KNN), I32),
        scratch_shapes=[
            pltpu.VMEM((MB, KNN), F32),
            pltpu.VMEM((MB, KNN), I32),
        ],
    )(jnp.asarray(_GRID_S), jnp.asarray(_GRID_S).astype(jnp.bfloat16),
      in_T, in_Tb)

    idx_km = nbr.T.reshape(-1)
    idx2d = idx_km.reshape(-1, SC_CH)
    G = _sc_gather(P, idx2d, M * KNN)

    G3 = G.reshape(KNN, M, HID)
    out = pl.pallas_call(
        _combine_body,
        grid=(M // CMB,),
        in_specs=[
            pl.BlockSpec((KNN, CMB, HID), lambda i: (0, i, 0)),
            pl.BlockSpec((CMB, HID), lambda i: (i, 0)),
            pl.BlockSpec((HID, COUT), lambda i: (0, 0)),
            pl.BlockSpec((1, COUT), lambda i: (0, 0)),
            pl.BlockSpec((COUT, HID), lambda i: (0, 0)),
            pl.BlockSpec((1, HID), lambda i: (0, 0)),
            pl.BlockSpec((HID, COUT), lambda i: (0, 0)),
            pl.BlockSpec((1, COUT), lambda i: (0, 0)),
        ],
        out_specs=pl.BlockSpec((CMB, COUT), lambda i: (i, 0)),
        out_shape=jax.ShapeDtypeStruct((M, COUT), F32),
    )(G3, Q, W_e2, b_e2.reshape(1, COUT), W_o1, b_o1.reshape(1, HID),
      W_o2, b_o2.reshape(1, COUT))

    return out.reshape(RES, RES, RES, COUT)
